# Initial kernel scaffold; baseline (speedup 1.0000x reference)
import functools

import numpy as np
import jax
import jax.numpy as jnp
from jax import lax
from jax.experimental import pallas as pl
from jax.experimental.pallas import tpu as pltpu
from jax.experimental.pallas import tpu_sc as plsc

_P = 0.5
_BS = 8
_C = 5
_H, _W = 64, 2048
_N = _H * _W
_ND = int(_N * _P)
_NQ = 4
_CH = _N // _NQ
_NW = _BS * _NQ
_LANES = 16


def _build_drop_index_table() -> np.ndarray:
    key = jax.random.key(42)
    buckets = []
    for b in range(_BS):
        kb = jax.random.fold_in(key, b + 1)
        perm = jax.random.permutation(kb, _N)
        drop = np.asarray(perm[:_ND])
        for q in range(_NQ):
            sel = drop[(drop >= q * _CH) & (drop < (q + 1) * _CH)] - q * _CH
            buckets.append(sel.astype(np.int32))
    maxc = max(len(s) for s in buckets)
    pad = -(-maxc // _LANES) * _LANES
    arr = np.empty((_NW, pad), np.int32)
    for i, s in enumerate(buckets):
        arr[i, : len(s)] = s
        arr[i, len(s):] = s[0]
    return arr


_IDX = _build_drop_index_table()
_PAD = _IDX.shape[1]


@functools.partial(
    pl.kernel,
    out_type=jax.ShapeDtypeStruct((_BS, _C, _N), jnp.float32),
    mesh=plsc.VectorSubcoreMesh(core_axis_name="c", subcore_axis_name="s"),
    scratch_types=[
        pltpu.VMEM((_PAD,), jnp.int32),
        pltpu.VMEM((_CH,), jnp.float32),
        pltpu.VMEM((_CH,), jnp.float32),
        pltpu.SemaphoreType.DMA,
        pltpu.SemaphoreType.DMA,
        pltpu.SemaphoreType.DMA,
        pltpu.SemaphoreType.DMA,
    ],
)
def _sc_drop(x_hbm, idx_hbm, out_hbm, idx_v, buf0, buf1,
             isem0, isem1, osem0, osem1):
    wid = lax.axis_index("s") * 2 + lax.axis_index("c")
    b = wid // _NQ
    q = lax.rem(wid, _NQ)
    base = q * _CH

    pltpu.sync_copy(idx_hbm.at[wid], idx_v)

    zeros = jnp.zeros((_LANES,), jnp.float32)

    def scatter_zeros(buf):
        def body(i, carry):
            iv = idx_v[pl.ds(i * _LANES, _LANES)]
            plsc.store_scatter(buf, [iv], zeros)
            return carry
        lax.fori_loop(0, _PAD // _LANES, body, 0)

    bufs = (buf0, buf1)
    isems = (isem0, isem1)
    osems = (osem0, osem1)
    pending_out = [None, None]

    pltpu.async_copy(x_hbm.at[b, 0, pl.ds(base, _CH)], buf0, isem0)
    for c in range(_C):
        cur = c % 2
        if c + 1 < _C:
            nxt = (c + 1) % 2
            if pending_out[nxt] is not None:
                pending_out[nxt].wait()
                pending_out[nxt] = None
            pltpu.async_copy(x_hbm.at[b, c + 1, pl.ds(base, _CH)],
                             bufs[nxt], isems[nxt])
        pltpu.make_async_copy(x_hbm.at[b, c, pl.ds(base, _CH)],
                              bufs[cur], isems[cur]).wait()
        scatter_zeros(bufs[cur])
        pending_out[cur] = pltpu.async_copy(
            bufs[cur], out_hbm.at[b, c, pl.ds(base, _CH)], osems[cur])
    for p in pending_out:
        if p is not None:
            p.wait()


def kernel(range_img):
    x = range_img.reshape(_BS, _C, _N)
    idx = jnp.asarray(_IDX)
    out = _sc_drop(x, idx)
    return out.reshape(range_img.shape)


# speedup vs baseline: 41.9628x; 41.9628x over previous
import functools

import numpy as np
import jax
import jax.numpy as jnp
from jax import lax
from jax.experimental import pallas as pl
from jax.experimental.pallas import tpu as pltpu
from jax.experimental.pallas import tpu_sc as plsc

_P = 0.5
_BS = 8
_C = 5
_H, _W = 64, 2048
_N = _H * _W
_ND = int(_N * _P)
_NQ = 4
_CH = _N // _NQ
_NW = _BS * _NQ
_LANES = 16


_U = np.uint32


def _tf2x32(k1, k2, x1, x2):
    rot0 = (13, 15, 26, 6)
    rot1 = (17, 29, 16, 24)
    ks0 = _U(k1)
    ks1 = _U(k2)
    ks2 = _U(ks0 ^ ks1 ^ _U(0x1BD11BDA))
    x = [(x1 + ks0).astype(_U), (x2 + ks1).astype(_U)]

    def rounds(x, rots):
        for r in rots:
            a = (x[0] + x[1]).astype(_U)
            b = ((x[1] << _U(r)) | (x[1] >> _U(32 - r))).astype(_U)
            x = [a, a ^ b]
        return x

    x = rounds(x, rot0); x = [(x[0] + ks1).astype(_U), (x[1] + ks2 + _U(1)).astype(_U)]
    x = rounds(x, rot1); x = [(x[0] + ks2).astype(_U), (x[1] + ks0 + _U(2)).astype(_U)]
    x = rounds(x, rot0); x = [(x[0] + ks0).astype(_U), (x[1] + ks1 + _U(3)).astype(_U)]
    x = rounds(x, rot1); x = [(x[0] + ks1).astype(_U), (x[1] + ks2 + _U(4)).astype(_U)]
    x = rounds(x, rot0); x = [(x[0] + ks2).astype(_U), (x[1] + ks0 + _U(5)).astype(_U)]
    return x


def _fold_in(key, data):
    o1, o2 = _tf2x32(key[0], key[1], np.array([0], _U), np.array([data], _U))
    return (o1[0], o2[0])


def _split2(key):
    b1, b2 = _tf2x32(key[0], key[1], np.array([0, 0], _U), np.array([0, 1], _U))
    return (b1[0], b2[0]), (b1[1], b2[1])


def _permutation(key, n):
    x = np.arange(n, dtype=np.int32)
    num_rounds = int(np.ceil(3 * np.log(n) / np.log(2**32 - 1)))
    for _ in range(num_rounds):
        key, sub = _split2(key)
        c1 = np.zeros(n, _U)
        c2 = np.arange(n, dtype=_U)
        b1, b2 = _tf2x32(sub[0], sub[1], c1, c2)
        x = x[np.argsort(b1 ^ b2, kind="stable")]
    return x


def _build_drop_index_table() -> np.ndarray:
    key42 = (_U(0), _U(42))
    buckets = []
    for b in range(_BS):
        perm = _permutation(_fold_in(key42, b + 1), _N)
        drop = perm[:_ND]
        for q in range(_NQ):
            sel = drop[(drop >= q * _CH) & (drop < (q + 1) * _CH)] - q * _CH
            buckets.append(sel.astype(np.int32))
    maxc = max(len(s) for s in buckets)
    pad = -(-maxc // _LANES) * _LANES
    arr = np.empty((_NW, pad), np.int32)
    for i, s in enumerate(buckets):
        arr[i, : len(s)] = s
        arr[i, len(s):] = s[0]
    return arr


_IDX = _build_drop_index_table()
_PAD = _IDX.shape[1]


@functools.partial(
    pl.kernel,
    out_type=jax.ShapeDtypeStruct((_BS * _C * _N,), jnp.float32),
    mesh=plsc.VectorSubcoreMesh(core_axis_name="c", subcore_axis_name="s",
                                num_cores=2, num_subcores=16),
    compiler_params=pltpu.CompilerParams(use_tc_tiling_on_sc=True,
                                         needs_layout_passes=False),
    scratch_types=[
        pltpu.VMEM((_PAD,), jnp.int32),
        pltpu.VMEM((_CH,), jnp.float32),
        pltpu.VMEM((_CH,), jnp.float32),
        pltpu.SemaphoreType.DMA,
        pltpu.SemaphoreType.DMA,
        pltpu.SemaphoreType.DMA,
        pltpu.SemaphoreType.DMA,
    ],
)
def _sc_drop(x_hbm, idx_hbm, out_hbm, idx_v, buf0, buf1,
             isem0, isem1, osem0, osem1):
    wid = lax.axis_index("s") * 2 + lax.axis_index("c")
    b = wid // _NQ
    q = lax.rem(wid, _NQ)
    base = q * _CH

    pltpu.sync_copy(idx_hbm.at[pl.ds(wid * _PAD, _PAD)], idx_v)

    def chunk_off(c):
        return (b * _C + c) * _N + base

    zeros = jnp.zeros((_LANES,), jnp.float32)

    def scatter_zeros(buf):
        def body(i, carry):
            iv = idx_v[pl.ds(i * _LANES, _LANES)]
            plsc.store_scatter(buf, [iv], zeros)
            return carry
        lax.fori_loop(0, _PAD // _LANES, body, 0)

    bufs = (buf0, buf1)
    isems = (isem0, isem1)
    osems = (osem0, osem1)
    pending_out = [None, None]

    pltpu.async_copy(x_hbm.at[pl.ds(chunk_off(0), _CH)], buf0, isem0)
    for c in range(_C):
        cur = c % 2
        if c + 1 < _C:
            nxt = (c + 1) % 2
            if pending_out[nxt] is not None:
                pending_out[nxt].wait()
                pending_out[nxt] = None
            pltpu.async_copy(x_hbm.at[pl.ds(chunk_off(c + 1), _CH)],
                             bufs[nxt], isems[nxt])
        pltpu.make_async_copy(x_hbm.at[pl.ds(chunk_off(c), _CH)],
                              bufs[cur], isems[cur]).wait()
        scatter_zeros(bufs[cur])
        pending_out[cur] = pltpu.async_copy(
            bufs[cur], out_hbm.at[pl.ds(chunk_off(c), _CH)], osems[cur])
    for p in pending_out:
        if p is not None:
            p.wait()


def kernel(range_img):
    x = range_img.reshape(_BS * _C * _N)
    idx = jnp.asarray(_IDX.reshape(-1))
    out = _sc_drop(x, idx)
    return out.reshape(range_img.shape)


# speedup vs baseline: 119.7880x; 2.8546x over previous
import functools

import numpy as np
import jax
import jax.numpy as jnp
from jax import lax
from jax.experimental import pallas as pl
from jax.experimental.pallas import tpu as pltpu
from jax.experimental.pallas import tpu_sc as plsc

_P = 0.5
_BS = 8
_C = 5
_H, _W = 64, 2048
_N = _H * _W
_ND = int(_N * _P)
_NQ = 4
_CH = _N // _NQ
_ROWS = _H // _NQ
_NW = _BS * _NQ
_LANES = 16
_UNROLL = 16


_U = np.uint32


def _tf2x32(k1, k2, x1, x2):
    rot0 = (13, 15, 26, 6)
    rot1 = (17, 29, 16, 24)
    ks0 = _U(k1)
    ks1 = _U(k2)
    ks2 = _U(ks0 ^ ks1 ^ _U(0x1BD11BDA))
    x = [(x1 + ks0).astype(_U), (x2 + ks1).astype(_U)]

    def rounds(x, rots):
        for r in rots:
            a = (x[0] + x[1]).astype(_U)
            b = ((x[1] << _U(r)) | (x[1] >> _U(32 - r))).astype(_U)
            x = [a, a ^ b]
        return x

    x = rounds(x, rot0); x = [(x[0] + ks1).astype(_U), (x[1] + ks2 + _U(1)).astype(_U)]
    x = rounds(x, rot1); x = [(x[0] + ks2).astype(_U), (x[1] + ks0 + _U(2)).astype(_U)]
    x = rounds(x, rot0); x = [(x[0] + ks0).astype(_U), (x[1] + ks1 + _U(3)).astype(_U)]
    x = rounds(x, rot1); x = [(x[0] + ks1).astype(_U), (x[1] + ks2 + _U(4)).astype(_U)]
    x = rounds(x, rot0); x = [(x[0] + ks2).astype(_U), (x[1] + ks0 + _U(5)).astype(_U)]
    return x


def _fold_in(key, data):
    o1, o2 = _tf2x32(key[0], key[1], np.array([0], _U), np.array([data], _U))
    return (o1[0], o2[0])


def _split2(key):
    b1, b2 = _tf2x32(key[0], key[1], np.array([0, 0], _U), np.array([0, 1], _U))
    return (b1[0], b2[0]), (b1[1], b2[1])


def _permutation(key, n):
    x = np.arange(n, dtype=np.int32)
    num_rounds = int(np.ceil(3 * np.log(n) / np.log(2**32 - 1)))
    for _ in range(num_rounds):
        key, sub = _split2(key)
        c1 = np.zeros(n, _U)
        c2 = np.arange(n, dtype=_U)
        b1, b2 = _tf2x32(sub[0], sub[1], c1, c2)
        x = x[np.argsort(b1 ^ b2, kind="stable")]
    return x


def _build_drop_index_table() -> np.ndarray:
    key42 = (_U(0), _U(42))
    buckets = []
    for b in range(_BS):
        perm = _permutation(_fold_in(key42, b + 1), _N)
        drop = perm[:_ND]
        for q in range(_NQ):
            sel = drop[(drop >= q * _CH) & (drop < (q + 1) * _CH)] - q * _CH
            buckets.append(sel.astype(np.int32))
    maxc = max(len(s) for s in buckets)
    pad = -(-maxc // (_LANES * _UNROLL)) * (_LANES * _UNROLL)
    arr = np.empty((_NW, pad), np.int32)
    for i, s in enumerate(buckets):
        arr[i, : len(s)] = s
        arr[i, len(s):] = s[0]
    return arr


_IDX = _build_drop_index_table()
_PAD = _IDX.shape[1]


@functools.partial(
    pl.kernel,
    out_type=jax.ShapeDtypeStruct((_BS, _C, _H, _W), jnp.float32),
    mesh=plsc.VectorSubcoreMesh(core_axis_name="c", subcore_axis_name="s",
                                num_cores=2, num_subcores=16),
    compiler_params=pltpu.CompilerParams(use_tc_tiling_on_sc=True,
                                         needs_layout_passes=False),
    scratch_types=[
        pltpu.VMEM((_PAD,), jnp.int32),
        pltpu.VMEM((_ROWS, _W), jnp.float32),
        pltpu.VMEM((_ROWS, _W), jnp.float32),
        pltpu.VMEM((_ROWS, _W), jnp.float32),
        pltpu.SemaphoreType.DMA,
        pltpu.SemaphoreType.DMA,
        pltpu.SemaphoreType.DMA,
        pltpu.SemaphoreType.DMA,
        pltpu.SemaphoreType.DMA,
        pltpu.SemaphoreType.DMA,
        pltpu.SemaphoreType.DMA,
    ],
)
def _sc_drop(x_hbm, idx_hbm, out_hbm, idx_v, buf0, buf1, buf2,
             isem0, isem1, isem2, osem0, osem1, osem2, xsem):
    wid = lax.axis_index("s") * 2 + lax.axis_index("c")
    b = wid // _NQ
    q = lax.rem(wid, _NQ)
    row0 = q * _ROWS

    def slab(ref, c):
        return ref.at[b, c, pl.ds(row0, _ROWS), :]

    zeros = jnp.zeros((_LANES,), jnp.float32)

    def scatter_zeros(buf):
        @plsc.parallel_loop(0, _PAD, _LANES, unroll=_UNROLL)
        def _(i):
            iv = idx_v[pl.ds(i, _LANES)]
            ivr = lax.shift_right_logical(iv, 11)
            ivc = lax.bitwise_and(iv, 2047)
            plsc.store_scatter(buf, [ivr, ivc], zeros)

    bufs = (buf0, buf1, buf2)
    isems = (isem0, isem1, isem2)
    osems = (osem0, osem1, osem2)
    pending_out = [None, None, None]

    pltpu.async_copy(slab(x_hbm, 0), buf0, isem0)
    pltpu.async_copy(slab(x_hbm, 1), buf1, isem1)
    idx_cp = pltpu.async_copy(idx_hbm.at[pl.ds(wid * _PAD, _PAD)], idx_v, xsem)
    idx_cp.wait()
    for c in range(_C):
        cur = c % 3
        if c + 2 < _C:
            nxt = (c + 2) % 3
            if pending_out[nxt] is not None:
                pending_out[nxt].wait()
                pending_out[nxt] = None
            pltpu.async_copy(slab(x_hbm, c + 2), bufs[nxt], isems[nxt])
        pltpu.make_async_copy(slab(x_hbm, c), bufs[cur], isems[cur]).wait()
        scatter_zeros(bufs[cur])
        pending_out[cur] = pltpu.async_copy(
            bufs[cur], slab(out_hbm, c), osems[cur])
    for p in pending_out:
        if p is not None:
            p.wait()


def kernel(range_img):
    idx = jnp.asarray(_IDX.reshape(-1))
    return _sc_drop(range_img, idx)
